# Initial kernel scaffold; baseline (speedup 1.0000x reference)
import jax
import jax.numpy as jnp
from jax import lax
from jax.experimental import pallas as pl
from jax.experimental.pallas import tpu as pltpu, tpu_sc as plsc

_EMBED_DIM = 64


def _make_gather(num_rows, batch_flat):
    info = plsc.get_sparse_core_info()
    nw = info.num_cores * info.num_subcores
    assert batch_flat % nw == 0
    b_per_w = batch_flat // nw
    chunk = 512
    while b_per_w % chunk:
        chunk //= 2
    n_chunks = b_per_w // chunk

    mesh = plsc.VectorSubcoreMesh(core_axis_name="c", subcore_axis_name="s")

    @pl.kernel(
        mesh=mesh,
        out_type=jax.ShapeDtypeStruct((batch_flat, _EMBED_DIM), jnp.float32),
        scratch_types=[
            pltpu.VMEM((b_per_w,), jnp.int32),
            pltpu.VMEM((chunk, _EMBED_DIM), jnp.float32),
            pltpu.SemaphoreType.DMA,
        ],
    )
    def gather_kernel(idx_hbm, table_hbm, out_hbm, idx_v, rows_v, sem):
        wid = lax.axis_index("s") * info.num_cores + lax.axis_index("c")
        base = wid * b_per_w
        pltpu.sync_copy(idx_hbm.at[pl.ds(base, b_per_w)], idx_v)

        def body(c, carry):
            off = c * chunk
            pltpu.async_copy(
                table_hbm.at[idx_v.at[pl.ds(off, chunk)]], rows_v, sem
            ).wait()
            pltpu.sync_copy(rows_v, out_hbm.at[pl.ds(base + off, chunk)])
            return carry

        lax.fori_loop(0, n_chunks, body, 0)

    return gather_kernel


def kernel(idx, table):
    b, h = idx.shape
    flat_idx = idx.reshape(-1).astype(jnp.int32)
    out = _make_gather(table.shape[0], b * h)(flat_idx, table)
    return out.reshape(b, h, _EMBED_DIM)


# speedup vs baseline: 1.8312x; 1.8312x over previous
import jax
import jax.numpy as jnp
from jax import lax
from jax.experimental import pallas as pl
from jax.experimental.pallas import tpu as pltpu, tpu_sc as plsc

_EMBED_DIM = 64


def _make_gather(num_rows, batch_flat):
    info = plsc.get_sparse_core_info()
    nw = info.num_cores * info.num_subcores
    assert batch_flat % nw == 0
    b_per_w = batch_flat // nw
    chunk = 512
    while b_per_w % chunk:
        chunk //= 2
    n_chunks = b_per_w // chunk

    mesh = plsc.VectorSubcoreMesh(core_axis_name="c", subcore_axis_name="s")

    @pl.kernel(
        mesh=mesh,
        out_type=jax.ShapeDtypeStruct((batch_flat, _EMBED_DIM), jnp.float32),
        scratch_types=[
            pltpu.VMEM((b_per_w,), jnp.int32),
            pltpu.VMEM((chunk, _EMBED_DIM), jnp.float32),
            pltpu.SemaphoreType.DMA,
        ],
        compiler_params=pltpu.CompilerParams(use_tc_tiling_on_sc=False),
    )
    def gather_kernel(idx_hbm, table_hbm, out_hbm, idx_v, rows_v, sem):
        wid = lax.axis_index("s") * info.num_cores + lax.axis_index("c")
        base = wid * b_per_w
        pltpu.sync_copy(idx_hbm.at[pl.ds(base, b_per_w)], idx_v)

        def body(c, carry):
            off = c * chunk
            pltpu.async_copy(
                table_hbm.at[idx_v.at[pl.ds(off, chunk)]], rows_v, sem
            ).wait()
            pltpu.sync_copy(rows_v, out_hbm.at[pl.ds(base + off, chunk)])
            return carry

        lax.fori_loop(0, n_chunks, body, 0)

    return gather_kernel


def kernel(idx, table):
    b, h = idx.shape
    flat_idx = idx.reshape(-1).astype(jnp.int32)
    out = _make_gather(table.shape[0], b * h)(flat_idx, table)
    return out.reshape(b, h, _EMBED_DIM)


# speedup vs baseline: 1.8756x; 1.0242x over previous
import jax
import jax.numpy as jnp
from jax import lax
from jax.experimental import pallas as pl
from jax.experimental.pallas import tpu as pltpu, tpu_sc as plsc

_EMBED_DIM = 64


def _make_gather(num_rows, batch_flat, chunk=128, nbuf=8):
    info = plsc.get_sparse_core_info()
    nw = info.num_cores * info.num_subcores
    assert batch_flat % nw == 0
    b_per_w = batch_flat // nw
    group = chunk * nbuf
    assert b_per_w % group == 0
    n_groups = b_per_w // group

    mesh = plsc.VectorSubcoreMesh(core_axis_name="c", subcore_axis_name="s")

    @pl.kernel(
        mesh=mesh,
        out_type=jax.ShapeDtypeStruct((batch_flat, _EMBED_DIM), jnp.float32),
        scratch_types=[
            pltpu.VMEM((b_per_w,), jnp.int32),
            [pltpu.VMEM((chunk, _EMBED_DIM), jnp.float32) for _ in range(nbuf)],
            [pltpu.SemaphoreType.DMA for _ in range(nbuf)],
            [pltpu.SemaphoreType.DMA for _ in range(nbuf)],
        ],
        compiler_params=pltpu.CompilerParams(use_tc_tiling_on_sc=False),
    )
    def gather_kernel(idx_hbm, table_hbm, out_hbm, idx_v, rows, gsems, wsems):
        wid = lax.axis_index("s") * info.num_cores + lax.axis_index("c")
        base = wid * b_per_w
        pltpu.sync_copy(idx_hbm.at[pl.ds(base, b_per_w)], idx_v)

        def body(g, carry):
            goff = g * group
            gh = [
                pltpu.async_copy(
                    table_hbm.at[idx_v.at[pl.ds(goff + b * chunk, chunk)]],
                    rows[b],
                    gsems[b],
                )
                for b in range(nbuf)
            ]
            wh = []
            for b in range(nbuf):
                gh[b].wait()
                wh.append(
                    pltpu.async_copy(
                        rows[b],
                        out_hbm.at[pl.ds(base + goff + b * chunk, chunk)],
                        wsems[b],
                    )
                )
            for b in range(nbuf):
                wh[b].wait()
            return carry

        lax.fori_loop(0, n_groups, body, 0)

    return gather_kernel


def kernel(idx, table):
    b, h = idx.shape
    flat_idx = idx.reshape(-1).astype(jnp.int32)
    out = _make_gather(table.shape[0], b * h)(flat_idx, table)
    return out.reshape(b, h, _EMBED_DIM)
